# baseline (speedup 1.0000x reference)
import functools
import math

import jax
import jax.numpy as jnp
from jax import lax
from jax.experimental import pallas as pl
from jax.experimental.pallas import tpu as pltpu
from jax.experimental.pallas import tpu_sc as plsc

M = 0.5
S = 64.0
COS_M = math.cos(M)
SIN_M = math.sin(M)
THRESHOLD = math.cos(math.pi - M)
MM = math.sin(math.pi - M) * M

B = 1024
C = 100000


_GATH = 32
_PREP_STEPS = B // _GATH


def _prep_body(lbl_sm, *refs):
    blocks = refs[:_GATH]
    ctm_o, ftls_o, t_o = refs[_GATH : _GATH + 3]
    tl_sc = refs[_GATH + 3]
    i = pl.program_id(0)
    sub_iota = lax.broadcasted_iota(jnp.int32, (8, 128), 0)
    lane_iota = lax.broadcasted_iota(jnp.int32, (8, 128), 1)
    acc = jnp.zeros((8, 128), jnp.float32)
    for k in range(_GATH):
        row = i * _GATH + k
        lbl = lbl_sm[row]
        m = (sub_iota == lbl % 8) & (lane_iota == row % 128)
        val = jnp.sum(jnp.where(m, blocks[k][...], 0.0))
        m2 = (sub_iota == row // 128) & (lane_iota == row % 128)
        acc = acc + jnp.where(m2, val, 0.0)
    tl_sc[...] = jnp.where(i == 0, acc, tl_sc[...] + acc)

    @pl.when(i == _PREP_STEPS - 1)
    def _():
        tl = jnp.clip(tl_sc[...], -1.0, 1.0)
        sin_theta = jnp.sqrt(1.0 - tl * tl)
        ctm = tl * COS_M - sin_theta * SIN_M
        ftl = jnp.where(tl > THRESHOLD, ctm, tl - MM)
        t = jnp.mean(tl) * 0.01
        ctm_o[...] = ctm
        ftls_o[...] = ftl * S
        t_o[...] = jnp.full((8, 128), t, jnp.float32)


def _make_block_spec(k):
    return pl.BlockSpec(
        (8, 128),
        lambda i, lbl: (lbl[i * _GATH + k] // 8, (i * _GATH + k) // 128),
    )


def _prep(ct_t, labels):
    grid_spec = pltpu.PrefetchScalarGridSpec(
        num_scalar_prefetch=1,
        grid=(_PREP_STEPS,),
        in_specs=[_make_block_spec(k) for k in range(_GATH)],
        out_specs=[
            pl.BlockSpec((8, 128), lambda i, lbl: (0, 0)),
            pl.BlockSpec((8, 128), lambda i, lbl: (0, 0)),
            pl.BlockSpec((8, 128), lambda i, lbl: (0, 0)),
        ],
        scratch_shapes=[pltpu.VMEM((8, 128), jnp.float32)],
    )
    return pl.pallas_call(
        _prep_body,
        grid_spec=grid_spec,
        out_shape=[
            jax.ShapeDtypeStruct((8, 128), jnp.float32),
            jax.ShapeDtypeStruct((8, 128), jnp.float32),
            jax.ShapeDtypeStruct((8, 128), jnp.float32),
        ],
    )(labels, *([ct_t] * _GATH))


_NC, _NS, _L = 2, 16, 16
_NW = _NC * _NS
_NPAIRS = C // 16
_NB_LO = _NPAIRS // _NW
_NB_EXTRA = _NPAIRS - _NB_LO * _NW
_KB = B // _L
_CR = 16


@functools.cache
def _dense_sc_kernel():
    @functools.partial(
        pl.kernel,
        out_type=jax.ShapeDtypeStruct((C, B), jnp.float32),
        mesh=plsc.VectorSubcoreMesh(core_axis_name="c", subcore_axis_name="s"),
        scratch_types=[
            pltpu.VMEM((2, _CR, B), jnp.float32),
            pltpu.VMEM((2, _CR, B), jnp.float32),
            pltpu.VMEM((8, 128), jnp.float32),
            pltpu.VMEM((8, 128), jnp.float32),
            pltpu.VMEM((8, 128), jnp.float32),
            pltpu.VMEM((8, 128), jnp.float32),
            pltpu.SemaphoreType.DMA,
            pltpu.SemaphoreType.DMA,
            pltpu.SemaphoreType.DMA,
            pltpu.SemaphoreType.DMA,
        ],
    )
    def _dense_sc(
        ct_hbm,
        ctm_hbm,
        ftls_hbm,
        lblf_hbm,
        t_hbm,
        out_hbm,
        ibuf,
        obuf,
        ctm_v,
        ftls_v,
        lblf_v,
        t_v,
        sem_i0,
        sem_i1,
        sem_o0,
        sem_o1,
    ):
        wid = lax.axis_index("s") * _NC + lax.axis_index("c")
        pltpu.sync_copy(ctm_hbm, ctm_v)
        pltpu.sync_copy(ftls_hbm, ftls_v)
        pltpu.sync_copy(lblf_hbm, lblf_v)
        pltpu.sync_copy(t_hbm, t_v)

        t_s = t_v[0, pl.ds(0, _L)]
        nb = _NB_LO + jnp.where(wid < _NB_EXTRA, 1, 0)

        sems_i = (sem_i0, sem_i1)
        sems_o = (sem_o0, sem_o1)

        def band_of(g):
            return wid + g * _NW

        def in_slice(g):
            return ct_hbm.at[pl.ds(band_of(g) * _CR, _CR), :]

        def out_slice(g):
            return out_hbm.at[pl.ds(band_of(g) * _CR, _CR), :]

        def compute(slot, g):
            j0 = band_of(g) * _CR
            jf = [jnp.full((_L,), (j0 + r).astype(jnp.float32)) for r in range(_CR)]

            @plsc.parallel_loop(0, _KB, step=1, unroll=4)
            def _(k):
                ks = k // 8
                ko = (k % 8) * _L
                ctm16 = ctm_v[ks, pl.ds(ko, _L)]
                ftls16 = ftls_v[ks, pl.ds(ko, _L)]
                lblf16 = lblf_v[ks, pl.ds(ko, _L)]
                for r in range(_CR):
                    x = ibuf[slot, r, pl.ds(k * _L, _L)]
                    x = jnp.minimum(jnp.maximum(x, -1.0), 1.0)
                    y = x * S
                    z = y * (x + t_s)
                    o = jnp.where(x > ctm16, z, y)
                    o = jnp.where(jf[r] == lblf16, ftls16, o)
                    obuf[slot, r, pl.ds(k * _L, _L)] = o

        def slot_body(g, slot):
            @pl.when(g + 1 < nb)
            def _():
                pltpu.async_copy(in_slice(g + 1), ibuf.at[1 - slot], sems_i[1 - slot])

            pltpu.make_async_copy(in_slice(g), ibuf.at[slot], sems_i[slot]).wait()

            @pl.when(g >= 2)
            def _():
                pltpu.make_async_copy(
                    obuf.at[slot], out_slice(g - 2), sems_o[slot]
                ).wait()

            compute(slot, g)
            pltpu.async_copy(obuf.at[slot], out_slice(g), sems_o[slot])

        pltpu.async_copy(in_slice(0), ibuf.at[0], sems_i[0])

        def ring_body(h, carry):
            g0 = h * 2

            @pl.when(g0 < nb)
            def _():
                slot_body(g0, 0)

            @pl.when(g0 + 1 < nb)
            def _():
                slot_body(g0 + 1, 1)

            return carry

        lax.fori_loop(0, (_NB_LO + 2) // 2, ring_body, 0)

        @pl.when(nb % 2 == 1)
        def _():
            pltpu.make_async_copy(obuf.at[0], out_slice(nb - 1), sems_o[0]).wait()

            @pl.when(nb >= 2)
            def _():
                pltpu.make_async_copy(obuf.at[1], out_slice(nb - 2), sems_o[1]).wait()

        @pl.when(nb % 2 == 0)
        def _():
            pltpu.make_async_copy(obuf.at[1], out_slice(nb - 1), sems_o[1]).wait()
            pltpu.make_async_copy(obuf.at[0], out_slice(nb - 2), sems_o[0]).wait()

    return _dense_sc


def kernel(cos_theta, labels):
    ct_t = cos_theta.T
    ctm8, ftls8, t8 = _prep(ct_t, labels)
    lblf8 = labels.astype(jnp.float32).reshape(8, 128)
    out_t = _dense_sc_kernel()(ct_t, ctm8, ftls8, lblf8, t8)
    return out_t.T


# speedup vs baseline: 1.1872x; 1.1872x over previous
import functools
import math

import jax
import jax.numpy as jnp
from jax import lax
from jax.experimental import pallas as pl
from jax.experimental.pallas import tpu as pltpu
from jax.experimental.pallas import tpu_sc as plsc

M = 0.5
S = 64.0
COS_M = math.cos(M)
SIN_M = math.sin(M)
THRESHOLD = math.cos(math.pi - M)
MM = math.sin(math.pi - M) * M

B = 1024
C = 100000


_GATH = 32
_PREP_STEPS = B // _GATH


def _prep_body(lbl_sm, *refs):
    blocks = refs[:_GATH]
    ctm_o, ftls_o, t_o = refs[_GATH : _GATH + 3]
    tl_sc = refs[_GATH + 3]
    i = pl.program_id(0)
    sub_iota = lax.broadcasted_iota(jnp.int32, (8, 128), 0)
    lane_iota = lax.broadcasted_iota(jnp.int32, (8, 128), 1)
    acc = jnp.zeros((8, 128), jnp.float32)
    for k in range(_GATH):
        row = i * _GATH + k
        lbl = lbl_sm[row]
        m = (sub_iota == lbl % 8) & (lane_iota == row % 128)
        val = jnp.sum(jnp.where(m, blocks[k][...], 0.0))
        m2 = (sub_iota == row // 128) & (lane_iota == row % 128)
        acc = acc + jnp.where(m2, val, 0.0)
    tl_sc[...] = jnp.where(i == 0, acc, tl_sc[...] + acc)

    @pl.when(i == _PREP_STEPS - 1)
    def _():
        tl = jnp.clip(tl_sc[...], -1.0, 1.0)
        sin_theta = jnp.sqrt(1.0 - tl * tl)
        ctm = tl * COS_M - sin_theta * SIN_M
        ftl = jnp.where(tl > THRESHOLD, ctm, tl - MM)
        t = jnp.mean(tl) * 0.01
        ctm_o[...] = ctm
        ftls_o[...] = ftl * S
        t_o[...] = jnp.full((8, 128), t, jnp.float32)


def _make_block_spec(k):
    return pl.BlockSpec(
        (8, 128),
        lambda i, lbl: (lbl[i * _GATH + k] // 8, (i * _GATH + k) // 128),
    )


def _prep(ct_t, labels):
    grid_spec = pltpu.PrefetchScalarGridSpec(
        num_scalar_prefetch=1,
        grid=(_PREP_STEPS,),
        in_specs=[_make_block_spec(k) for k in range(_GATH)],
        out_specs=[
            pl.BlockSpec((8, 128), lambda i, lbl: (0, 0)),
            pl.BlockSpec((8, 128), lambda i, lbl: (0, 0)),
            pl.BlockSpec((8, 128), lambda i, lbl: (0, 0)),
        ],
        scratch_shapes=[pltpu.VMEM((8, 128), jnp.float32)],
    )
    return pl.pallas_call(
        _prep_body,
        grid_spec=grid_spec,
        out_shape=[
            jax.ShapeDtypeStruct((8, 128), jnp.float32),
            jax.ShapeDtypeStruct((8, 128), jnp.float32),
            jax.ShapeDtypeStruct((8, 128), jnp.float32),
        ],
    )(labels, *([ct_t] * _GATH))


_NC, _NS, _L = 2, 16, 16
_NW = _NC * _NS
_NBANDS = C // 8
_NB_LO = _NBANDS // _NW
_NB_EXTRA = _NBANDS - _NB_LO * _NW
_KB = B // _L


@functools.cache
def _dense_sc_kernel():
    @functools.partial(
        pl.kernel,
        out_type=jax.ShapeDtypeStruct((C, B), jnp.float32),
        mesh=plsc.VectorSubcoreMesh(core_axis_name="c", subcore_axis_name="s"),
        scratch_types=[
            pltpu.VMEM((3, 8, B), jnp.float32),
            pltpu.VMEM((3, 8, B), jnp.float32),
            pltpu.VMEM((8, 128), jnp.float32),
            pltpu.VMEM((8, 128), jnp.float32),
            pltpu.VMEM((8, 128), jnp.float32),
            pltpu.VMEM((8, 128), jnp.float32),
            pltpu.SemaphoreType.DMA,
            pltpu.SemaphoreType.DMA,
            pltpu.SemaphoreType.DMA,
            pltpu.SemaphoreType.DMA,
            pltpu.SemaphoreType.DMA,
            pltpu.SemaphoreType.DMA,
        ],
    )
    def _dense_sc(
        ct_hbm,
        ctm_hbm,
        ftls_hbm,
        lblf_hbm,
        t_hbm,
        out_hbm,
        ibuf,
        obuf,
        ctm_v,
        ftls_v,
        lblf_v,
        t_v,
        sem_i0,
        sem_i1,
        sem_i2,
        sem_o0,
        sem_o1,
        sem_o2,
    ):
        wid = lax.axis_index("s") * _NC + lax.axis_index("c")
        pltpu.sync_copy(ctm_hbm, ctm_v)
        pltpu.sync_copy(ftls_hbm, ftls_v)
        pltpu.sync_copy(lblf_hbm, lblf_v)
        pltpu.sync_copy(t_hbm, t_v)

        t_s = t_v[0, pl.ds(0, _L)]
        nb = _NB_LO + jnp.where(wid < _NB_EXTRA, 1, 0)

        sems_i = (sem_i0, sem_i1, sem_i2)
        sems_o = (sem_o0, sem_o1, sem_o2)

        def band_of(g):
            return wid + g * _NW

        def in_slice(g):
            return ct_hbm.at[pl.ds(band_of(g) * 8, 8), :]

        def out_slice(g):
            return out_hbm.at[pl.ds(band_of(g) * 8, 8), :]

        def compute(slot, g):
            j0 = band_of(g) * 8
            jf = [jnp.full((_L,), (j0 + r).astype(jnp.float32)) for r in range(8)]

            @plsc.parallel_loop(0, _KB, step=1, unroll=4)
            def _(k):
                ks = k // 8
                ko = (k % 8) * _L
                ctm16 = ctm_v[ks, pl.ds(ko, _L)]
                ftls16 = ftls_v[ks, pl.ds(ko, _L)]
                lblf16 = lblf_v[ks, pl.ds(ko, _L)]
                for r in range(8):
                    x = ibuf[slot, r, pl.ds(k * _L, _L)]
                    x = jnp.minimum(jnp.maximum(x, -1.0), 1.0)
                    y = x * S
                    z = y * (x + t_s)
                    o = jnp.where(x > ctm16, z, y)
                    o = jnp.where(jf[r] == lblf16, ftls16, o)
                    obuf[slot, r, pl.ds(k * _L, _L)] = o

        def slot_body(g, slot):
            nslot = (slot + 2) % 3

            @pl.when(g + 2 < nb)
            def _():
                pltpu.async_copy(in_slice(g + 2), ibuf.at[nslot], sems_i[nslot])

            pltpu.make_async_copy(in_slice(g), ibuf.at[slot], sems_i[slot]).wait()

            @pl.when(g >= 3)
            def _():
                pltpu.make_async_copy(
                    obuf.at[slot], out_slice(g - 3), sems_o[slot]
                ).wait()

            compute(slot, g)
            pltpu.async_copy(obuf.at[slot], out_slice(g), sems_o[slot])

        pltpu.async_copy(in_slice(0), ibuf.at[0], sems_i[0])

        @pl.when(nb >= 2)
        def _():
            pltpu.async_copy(in_slice(1), ibuf.at[1], sems_i[1])

        def ring_body(h, carry):
            g0 = h * 3
            for d in range(3):
                @pl.when(g0 + d < nb)
                def _(d=d):
                    slot_body(g0 + d, d)
            return carry

        lax.fori_loop(0, (_NB_LO + 3) // 3, ring_body, 0)

        for m in (0, 1):
            @pl.when(nb % 3 == m)
            def _(m=m):
                for d in range(3):
                    slot = (m - 1 - d) % 3
                    pltpu.make_async_copy(
                        obuf.at[slot], out_slice(nb - 1 - d), sems_o[slot]
                    ).wait()

    return _dense_sc


def kernel(cos_theta, labels):
    ct_t = cos_theta.T
    ctm8, ftls8, t8 = _prep(ct_t, labels)
    lblf8 = labels.astype(jnp.float32).reshape(8, 128)
    out_t = _dense_sc_kernel()(ct_t, ctm8, ftls8, lblf8, t8)
    return out_t.T
